# baseline (speedup 1.0000x reference)
import functools

import jax
import jax.numpy as jnp
from jax import lax
from jax.experimental import pallas as pl
from jax.experimental.pallas import tpu as pltpu
from jax.experimental.pallas import tpu_sc as plsc

L = 16
REP_SET = ()


@functools.partial(jax.jit, static_argnames=("batch", "ncols", "num_trees", "depth"))
def _traverse(x_t, root_nodes, root_biases, leaf_flat, batch, ncols,
              num_trees, depth, *tables):
    feats = tables[: depth - 1]
    thrs = tables[depth - 1:]
    n_leaves = 1 << depth

    info = plsc.get_sparse_core_info()
    nc, ns = info.num_cores, info.num_subcores
    nw = nc * ns
    rows = batch // nw
    ngrp = rows // L
    nblk = num_trees // L

    rep = [i in REP_SET for i in range(1, depth)]
    sizes = [L * (1 << i) * (L if rep[i - 1] else 1) for i in range(1, depth)]
    offs = [0]
    for s in sizes[:-1]:
        offs.append(offs[-1] + s)
    tab_total = offs[-1] + sizes[-1]
    leaf_blk = L * n_leaves

    mesh = plsc.VectorSubcoreMesh(core_axis_name="c", subcore_axis_name="s")

    def body(x_hbm, roots_hbm, biases_hbm, leaf_hbm, *rest):
        f_hbm = rest[: depth - 1]
        t_hbm = rest[depth - 1: 2 * (depth - 1)]
        (out_hbm, x_v, roots_v, biases_v, feats_vA, thrs_vA, leaf_vA,
         feats_vB, thrs_vB, leaf_vB, out_vA, out_vB, semA, semB,
         sem_out) = rest[2 * (depth - 1):]

        wid = lax.axis_index("s") * nc + lax.axis_index("c")
        b0 = wid * rows
        pltpu.sync_copy(x_hbm.at[:, pl.ds(b0, rows)], x_v)
        pltpu.sync_copy(roots_hbm, roots_v)
        pltpu.sync_copy(biases_hbm, biases_v)

        lane = lax.iota(jnp.int32, L)

        def fire(blk, fv, tv, lv, s):
            t0 = blk * L
            for i in range(depth - 1):
                span = (1 << (i + 1)) * (L if rep[i] else 1)
                pltpu.async_copy(f_hbm[i].at[pl.ds(t0 * span, sizes[i])],
                                 fv.at[pl.ds(offs[i], sizes[i])], s)
                pltpu.async_copy(t_hbm[i].at[pl.ds(t0 * span, sizes[i])],
                                 tv.at[pl.ds(offs[i], sizes[i])], s)
            pltpu.async_copy(leaf_hbm.at[pl.ds(t0 * n_leaves, leaf_blk)], lv, s)

        def drain(fv, tv, lv, s):
            for i in range(depth - 1):
                pltpu.make_async_copy(f_hbm[i].at[pl.ds(0, sizes[i])],
                                      fv.at[pl.ds(offs[i], sizes[i])], s).wait()
                pltpu.make_async_copy(t_hbm[i].at[pl.ds(0, sizes[i])],
                                      tv.at[pl.ds(offs[i], sizes[i])], s).wait()
            pltpu.make_async_copy(leaf_hbm.at[pl.ds(0, leaf_blk)], lv, s).wait()

        def compute(blk, feats_v, thrs_v, leaf_v, out_v):
            t0 = blk * L
            rv = roots_v[pl.ds(t0, L)]
            bv = biases_v[pl.ds(t0, L)]
            roots = []
            for tl in range(L):
                roots.append((lax.broadcast(rv[tl], (L,)),
                              lax.broadcast(bv[tl], (L,))))

            @plsc.parallel_loop(0, ngrp, 1, unroll=1)
            def grp_body(g):
                rowvec = g * L + lane
                leaves = []
                for tl in range(L):
                    rn, rb = roots[tl]
                    xv = plsc.load_gather(x_v, [rn, rowvec])
                    prev = (xv >= rb).astype(jnp.int32)
                    for i in range(1, depth):
                        if rep[i - 1]:
                            base = offs[i - 1] + tl * (1 << i) * L
                            idx = prev * L + lane + base
                        else:
                            idx = prev + (offs[i - 1] + tl * (1 << i))
                        fi = plsc.load_gather(feats_v, [idx])
                        th = plsc.load_gather(thrs_v, [idx])
                        xv = plsc.load_gather(x_v, [fi, rowvec])
                        prev = prev * 2 + (xv >= th).astype(jnp.int32)
                    leaves.append(
                        plsc.load_gather(leaf_v, [prev + tl * n_leaves]))
                for tl in range(L):
                    out_v[tl, pl.ds(g * L, L)] = leaves[tl]

            pltpu.async_copy(out_v,
                             out_hbm.at[pl.ds(t0, L), pl.ds(b0, rows)],
                             sem_out)

        def drain_out(out_v):
            pltpu.make_async_copy(
                out_v, out_hbm.at[pl.ds(0, L), pl.ds(b0, rows)],
                sem_out).wait()

        npair = nblk // 2
        fire(0, feats_vA, thrs_vA, leaf_vA, semA)

        def pair_body(k, carry):
            fire(2 * k + 1, feats_vB, thrs_vB, leaf_vB, semB)
            drain(feats_vA, thrs_vA, leaf_vA, semA)
            compute(2 * k, feats_vA, thrs_vA, leaf_vA, out_vA)

            @pl.when(k < npair - 1)
            def _prefetch_even():
                fire(2 * k + 2, feats_vA, thrs_vA, leaf_vA, semA)

            drain(feats_vB, thrs_vB, leaf_vB, semB)
            compute(2 * k + 1, feats_vB, thrs_vB, leaf_vB, out_vB)
            drain_out(out_vA)
            drain_out(out_vB)
            return carry

        lax.fori_loop(0, npair, pair_body, 0)

    run = pl.kernel(
        body,
        out_type=jax.ShapeDtypeStruct((num_trees, batch), jnp.float32),
        mesh=mesh,
        compiler_params=pltpu.CompilerParams(needs_layout_passes=False),
        scratch_types=[
            pltpu.VMEM((ncols, rows), jnp.float32),
            pltpu.VMEM((num_trees,), jnp.int32),
            pltpu.VMEM((num_trees,), jnp.float32),
            pltpu.VMEM((tab_total,), jnp.int32),
            pltpu.VMEM((tab_total,), jnp.float32),
            pltpu.VMEM((leaf_blk,), jnp.float32),
            pltpu.VMEM((tab_total,), jnp.int32),
            pltpu.VMEM((tab_total,), jnp.float32),
            pltpu.VMEM((leaf_blk,), jnp.float32),
            pltpu.VMEM((L, rows), jnp.float32),
            pltpu.VMEM((L, rows), jnp.float32),
            pltpu.SemaphoreType.DMA,
            pltpu.SemaphoreType.DMA,
            pltpu.SemaphoreType.DMA,
        ],
    )
    return run(x_t, root_nodes, root_biases, leaf_flat, *feats, *thrs)


def _lane_replicate(a):
    return jnp.broadcast_to(a[:, None], (a.shape[0], L)).reshape(-1)


def kernel(x, root_nodes, root_biases, tree_indices, level_feature_ids,
           level_thresholds, leaf_nodes):
    del tree_indices
    batch, ncols = x.shape
    num_trees = root_nodes.shape[0]
    depth = len(level_feature_ids) + 1
    n_classes = leaf_nodes.shape[1]
    feats = [_lane_replicate(f) if 1 + i in REP_SET else f
             for i, f in enumerate(level_feature_ids)]
    thrs = [_lane_replicate(t) if 1 + i in REP_SET else t
            for i, t in enumerate(level_thresholds)]
    out = _traverse(x.T, root_nodes, root_biases,
                    leaf_nodes.reshape(-1), batch, ncols, num_trees, depth,
                    *feats, *thrs)
    return out.T.reshape(batch, num_trees, n_classes)


# speedup vs baseline: 1.0223x; 1.0223x over previous
import functools

import jax
import jax.numpy as jnp
from jax import lax
from jax.experimental import pallas as pl
from jax.experimental.pallas import tpu as pltpu
from jax.experimental.pallas import tpu_sc as plsc

L = 16
REP_SET = ()


@functools.partial(jax.jit, static_argnames=("batch", "ncols", "num_trees", "depth"))
def _traverse(x_t, root_nodes, root_biases, leaf_flat, batch, ncols,
              num_trees, depth, *tables):
    feats = tables[: depth - 1]
    thrs = tables[depth - 1:]
    n_leaves = 1 << depth

    info = plsc.get_sparse_core_info()
    nc, ns = info.num_cores, info.num_subcores
    nw = nc * ns
    rows = batch // nw
    ngrp = rows // L
    nblk = num_trees // L

    rep = [i in REP_SET for i in range(1, depth)]
    sizes = [L * (1 << i) * (L if rep[i - 1] else 1) for i in range(1, depth)]
    offs = [0]
    for s in sizes[:-1]:
        offs.append(offs[-1] + s)
    tab_total = offs[-1] + sizes[-1]
    leaf_blk = L * n_leaves

    mesh = plsc.VectorSubcoreMesh(core_axis_name="c", subcore_axis_name="s")

    def body(x_hbm, roots_hbm, biases_hbm, leaf_hbm, *rest):
        f_hbm = rest[: depth - 1]
        t_hbm = rest[depth - 1: 2 * (depth - 1)]
        (out_hbm, x_v, roots_v, biases_v, feats_vA, thrs_vA, leaf_vA,
         feats_vB, thrs_vB, leaf_vB, out_vA, out_vB, semA, semB,
         sem_out) = rest[2 * (depth - 1):]

        wid = lax.axis_index("s") * nc + lax.axis_index("c")
        b0 = wid * rows
        pltpu.sync_copy(x_hbm.at[:, pl.ds(b0, rows)], x_v)
        pltpu.sync_copy(roots_hbm, roots_v)
        pltpu.sync_copy(biases_hbm, biases_v)

        lane = lax.iota(jnp.int32, L)

        def fire(blk, fv, tv, lv, s):
            t0 = blk * L
            for i in range(depth - 1):
                span = (1 << (i + 1)) * (L if rep[i] else 1)
                pltpu.async_copy(f_hbm[i].at[pl.ds(t0 * span, sizes[i])],
                                 fv.at[pl.ds(offs[i], sizes[i])], s)
                pltpu.async_copy(t_hbm[i].at[pl.ds(t0 * span, sizes[i])],
                                 tv.at[pl.ds(offs[i], sizes[i])], s)
            pltpu.async_copy(leaf_hbm.at[pl.ds(t0 * n_leaves, leaf_blk)], lv, s)

        def drain(fv, tv, lv, s):
            for i in range(depth - 1):
                pltpu.make_async_copy(f_hbm[i].at[pl.ds(0, sizes[i])],
                                      fv.at[pl.ds(offs[i], sizes[i])], s).wait()
                pltpu.make_async_copy(t_hbm[i].at[pl.ds(0, sizes[i])],
                                      tv.at[pl.ds(offs[i], sizes[i])], s).wait()
            pltpu.make_async_copy(leaf_hbm.at[pl.ds(0, leaf_blk)], lv, s).wait()

        def compute(blk, feats_v, thrs_v, leaf_v, out_v):
            t0 = blk * L
            roots = []
            for tl in range(L):
                tvec = jnp.full((L,), t0 + tl, jnp.int32)
                roots.append((plsc.load_gather(roots_v, [tvec]),
                              plsc.load_gather(biases_v, [tvec])))

            @plsc.parallel_loop(0, ngrp, 1, unroll=1)
            def grp_body(g):
                rowvec = g * L + lane
                leaves = []
                for tl in range(L):
                    rn, rb = roots[tl]
                    xv = plsc.load_gather(x_v, [rn, rowvec])
                    prev = (xv >= rb).astype(jnp.int32)
                    for i in range(1, depth):
                        if rep[i - 1]:
                            base = offs[i - 1] + tl * (1 << i) * L
                            idx = prev * L + lane + base
                        else:
                            idx = prev + (offs[i - 1] + tl * (1 << i))
                        fi = plsc.load_gather(feats_v, [idx])
                        th = plsc.load_gather(thrs_v, [idx])
                        xv = plsc.load_gather(x_v, [fi, rowvec])
                        prev = prev * 2 + (xv >= th).astype(jnp.int32)
                    leaves.append(
                        plsc.load_gather(leaf_v, [prev + tl * n_leaves]))
                for tl in range(L):
                    out_v[tl, pl.ds(g * L, L)] = leaves[tl]

            pltpu.async_copy(out_v,
                             out_hbm.at[pl.ds(t0, L), pl.ds(b0, rows)],
                             sem_out)

        def drain_out(out_v):
            pltpu.make_async_copy(
                out_v, out_hbm.at[pl.ds(0, L), pl.ds(b0, rows)],
                sem_out).wait()

        npair = nblk // 2
        fire(0, feats_vA, thrs_vA, leaf_vA, semA)

        def pair_body(k, carry):
            fire(2 * k + 1, feats_vB, thrs_vB, leaf_vB, semB)
            drain(feats_vA, thrs_vA, leaf_vA, semA)
            compute(2 * k, feats_vA, thrs_vA, leaf_vA, out_vA)

            @pl.when(k < npair - 1)
            def _prefetch_even():
                fire(2 * k + 2, feats_vA, thrs_vA, leaf_vA, semA)

            drain(feats_vB, thrs_vB, leaf_vB, semB)
            compute(2 * k + 1, feats_vB, thrs_vB, leaf_vB, out_vB)
            drain_out(out_vA)
            drain_out(out_vB)
            return carry

        lax.fori_loop(0, npair, pair_body, 0)

    run = pl.kernel(
        body,
        out_type=jax.ShapeDtypeStruct((num_trees, batch), jnp.float32),
        mesh=mesh,
        compiler_params=pltpu.CompilerParams(needs_layout_passes=False),
        scratch_types=[
            pltpu.VMEM((ncols, rows), jnp.float32),
            pltpu.VMEM((num_trees,), jnp.int32),
            pltpu.VMEM((num_trees,), jnp.float32),
            pltpu.VMEM((tab_total,), jnp.int32),
            pltpu.VMEM((tab_total,), jnp.float32),
            pltpu.VMEM((leaf_blk,), jnp.float32),
            pltpu.VMEM((tab_total,), jnp.int32),
            pltpu.VMEM((tab_total,), jnp.float32),
            pltpu.VMEM((leaf_blk,), jnp.float32),
            pltpu.VMEM((L, rows), jnp.float32),
            pltpu.VMEM((L, rows), jnp.float32),
            pltpu.SemaphoreType.DMA,
            pltpu.SemaphoreType.DMA,
            pltpu.SemaphoreType.DMA,
        ],
    )
    return run(x_t, root_nodes, root_biases, leaf_flat, *feats, *thrs)


def _lane_replicate(a):
    return jnp.broadcast_to(a[:, None], (a.shape[0], L)).reshape(-1)


def kernel(x, root_nodes, root_biases, tree_indices, level_feature_ids,
           level_thresholds, leaf_nodes):
    del tree_indices
    batch, ncols = x.shape
    num_trees = root_nodes.shape[0]
    depth = len(level_feature_ids) + 1
    n_classes = leaf_nodes.shape[1]
    feats = [_lane_replicate(f) if 1 + i in REP_SET else f
             for i, f in enumerate(level_feature_ids)]
    thrs = [_lane_replicate(t) if 1 + i in REP_SET else t
            for i, t in enumerate(level_thresholds)]
    out = _traverse(x.T, root_nodes, root_biases,
                    leaf_nodes.reshape(-1), batch, ncols, num_trees, depth,
                    *feats, *thrs)
    return out.T.reshape(batch, num_trees, n_classes)
